# Initial kernel scaffold; baseline (speedup 1.0000x reference)
import jax
import jax.numpy as jnp
from jax.experimental import pallas as pl


def kernel(student_output):
    raise NotImplementedError("write your pallas kernel here")


# speedup vs baseline: 20.3101x; 20.3101x over previous
import jax
import jax.numpy as jnp
from jax.experimental import pallas as pl
from jax.experimental.pallas import tpu as pltpu

_B = 8192
_D = 512
_BM = 512
_EPS = 1e-8
_NEG = -1e30


def _normalize_kernel(x_ref, o_ref):
    x = x_ref[...]
    n = jnp.sqrt(jnp.sum(x * x, axis=1, keepdims=True))
    o_ref[...] = x / (n + _EPS)


def _koleo_kernel(xr_ref, xa_ref, o_ref):
    i = pl.program_id(0)
    xr = xr_ref[...]
    xa = xa_ref[...]
    dots = jax.lax.dot_general(
        xr, xa, (((1,), (1,)), ((), ())), preferred_element_type=jnp.float32
    )
    col = jax.lax.broadcasted_iota(jnp.int32, dots.shape, 1)
    row_g = i * _BM + jax.lax.broadcasted_iota(jnp.int32, dots.shape, 0)
    dots = jnp.where(col == row_g, -1.0, dots)
    m1 = jnp.max(dots, axis=1, keepdims=True)
    first = jnp.min(jnp.where(dots == m1, col, _B), axis=1, keepdims=True)
    m2 = jnp.max(jnp.where(col == first, _NEG, dots), axis=1, keepdims=True)
    v = jnp.concatenate([m1, m2], axis=1)
    dist = jnp.sqrt(jnp.maximum(2.0 - 2.0 * v, 0.0)) + _EPS
    s = jnp.sum(jnp.log(dist + _EPS))

    @pl.when(i == 0)
    def _init():
        o_ref[0, 0] = 0.0

    o_ref[0, 0] += -s / (2.0 * _B)


def kernel(student_output):
    xn = pl.pallas_call(
        _normalize_kernel,
        grid=(_B // _BM,),
        in_specs=[pl.BlockSpec((_BM, _D), lambda i: (i, 0))],
        out_specs=pl.BlockSpec((_BM, _D), lambda i: (i, 0)),
        out_shape=jax.ShapeDtypeStruct((_B, _D), jnp.float32),
    )(student_output)

    loss = pl.pallas_call(
        _koleo_kernel,
        grid=(_B // _BM,),
        in_specs=[
            pl.BlockSpec((_BM, _D), lambda i: (i, 0)),
            pl.BlockSpec((_B, _D), lambda i: (0, 0)),
        ],
        out_specs=pl.BlockSpec(memory_space=pltpu.SMEM),
        out_shape=jax.ShapeDtypeStruct((1, 1), jnp.float32),
    )(xn, xn)
    return loss[0, 0]


# speedup vs baseline: 22.2773x; 1.0969x over previous
import jax
import jax.numpy as jnp
from jax.experimental import pallas as pl
from jax.experimental.pallas import tpu as pltpu

_B = 8192
_D = 512
_BM = 512
_EPS = 1e-8
_NEG = -1e30


def _koleo_kernel(x_ref, o_ref, xn_ref):
    i = pl.program_id(0)

    @pl.when(i == 0)
    def _init():
        x = x_ref[...]
        n = jnp.sqrt(jnp.sum(x * x, axis=1, keepdims=True))
        xn_ref[...] = x / (n + _EPS)
        o_ref[0, 0] = 0.0

    xr = xn_ref[pl.ds(i * _BM, _BM), :]
    xa = xn_ref[...]
    dots = jax.lax.dot_general(
        xr, xa, (((1,), (1,)), ((), ())), preferred_element_type=jnp.float32
    )
    col = jax.lax.broadcasted_iota(jnp.int32, dots.shape, 1)
    row_g = i * _BM + jax.lax.broadcasted_iota(jnp.int32, dots.shape, 0)
    dots = jnp.where(col == row_g, -1.0, dots)
    m1 = jnp.max(dots, axis=1, keepdims=True)
    first = jnp.min(jnp.where(dots == m1, col, _B), axis=1, keepdims=True)
    m2 = jnp.max(jnp.where(col == first, _NEG, dots), axis=1, keepdims=True)
    v = jnp.concatenate([m1, m2], axis=1)
    dist = jnp.sqrt(jnp.maximum(2.0 - 2.0 * v, 0.0)) + _EPS
    s = jnp.sum(jnp.log(dist + _EPS))
    o_ref[0, 0] += -s / (2.0 * _B)


def kernel(student_output):
    loss = pl.pallas_call(
        _koleo_kernel,
        grid=(_B // _BM,),
        in_specs=[pl.BlockSpec((_B, _D), lambda i: (0, 0))],
        out_specs=pl.BlockSpec(memory_space=pltpu.SMEM),
        out_shape=jax.ShapeDtypeStruct((1, 1), jnp.float32),
        scratch_shapes=[pltpu.VMEM((_B, _D), jnp.float32)],
    )(student_output)
    return loss[0, 0]


# speedup vs baseline: 23.6199x; 1.0603x over previous
import jax
import jax.numpy as jnp
from jax.experimental import pallas as pl
from jax.experimental.pallas import tpu as pltpu

_B = 8192
_D = 512
_BM = 512
_BN = 2048
_GI = _B // _BM
_GJ = _B // _BN
_EPS = 1e-8
_NEG = -1e30


def _koleo_kernel(x_ref, o_ref, xn_ref, m1_ref, m2_ref):
    i = pl.program_id(0)
    j = pl.program_id(1)

    @pl.when(jnp.logical_and(i == 0, j == 0))
    def _init():
        x = x_ref[...]
        n = jnp.sqrt(jnp.sum(x * x, axis=1, keepdims=True))
        xn_ref[...] = x / (n + _EPS)
        o_ref[0, 0] = 0.0

    xr = xn_ref[pl.ds(i * _BM, _BM), :]
    xa = xn_ref[pl.ds(j * _BN, _BN), :]
    dots = jax.lax.dot_general(
        xr, xa, (((1,), (1,)), ((), ())), preferred_element_type=jnp.float32
    )

    @pl.when(i // (_BN // _BM) == j)
    def _mask_diag():
        col = j * _BN + jax.lax.broadcasted_iota(jnp.int32, dots.shape, 1)
        row = i * _BM + jax.lax.broadcasted_iota(jnp.int32, dots.shape, 0)
        masked = jnp.where(col == row, -1.0, dots)
        m1_t = jnp.max(masked, axis=1, keepdims=True)
        m2_t = jnp.max(jnp.where(masked == m1_t, _NEG, masked), axis=1, keepdims=True)
        _merge(j, m1_ref, m2_ref, m1_t, m2_t)

    @pl.when(i // (_BN // _BM) != j)
    def _no_mask():
        m1_t = jnp.max(dots, axis=1, keepdims=True)
        m2_t = jnp.max(jnp.where(dots == m1_t, _NEG, dots), axis=1, keepdims=True)
        _merge(j, m1_ref, m2_ref, m1_t, m2_t)

    @pl.when(j == _GJ - 1)
    def _finish():
        v = jnp.concatenate([m1_ref[...], m2_ref[...]], axis=1)
        dist = jnp.sqrt(jnp.maximum(2.0 - 2.0 * v, 0.0)) + _EPS
        s = jnp.sum(jnp.log(dist + _EPS))
        o_ref[0, 0] += -s / (2.0 * _B)


def _merge(j, m1_ref, m2_ref, m1_t, m2_t):
    @pl.when(j == 0)
    def _():
        m1_ref[...] = m1_t
        m2_ref[...] = m2_t

    @pl.when(j != 0)
    def _():
        m1 = m1_ref[...]
        m2 = m2_ref[...]
        m1_ref[...] = jnp.maximum(m1, m1_t)
        m2_ref[...] = jnp.maximum(jnp.minimum(m1, m1_t), jnp.maximum(m2, m2_t))


def kernel(student_output):
    loss = pl.pallas_call(
        _koleo_kernel,
        grid=(_GI, _GJ),
        in_specs=[pl.BlockSpec((_B, _D), lambda i, j: (0, 0))],
        out_specs=pl.BlockSpec(memory_space=pltpu.SMEM),
        out_shape=jax.ShapeDtypeStruct((1, 1), jnp.float32),
        scratch_shapes=[
            pltpu.VMEM((_B, _D), jnp.float32),
            pltpu.VMEM((_BM, 1), jnp.float32),
            pltpu.VMEM((_BM, 1), jnp.float32),
        ],
    )(student_output)
    return loss[0, 0]
